# baseline (speedup 1.0000x reference)
import functools

import jax
import jax.numpy as jnp
from jax import lax
from jax.experimental import pallas as pl
from jax.experimental.pallas import tpu as pltpu
from jax.experimental.pallas import tpu_sc as plsc

NC = 2
NS = 16
NW = NC * NS
L = 16

_B = 2048
_STEPS = _B // L
_UNROLL = 8


@functools.cache
def _build(n_vars, n_constrs, nnz):
    tpc = n_constrs // NW
    rs = tpc + 1
    mesh = plsc.VectorSubcoreMesh(core_axis_name="c", subcore_axis_name="s")

    @functools.partial(
        pl.kernel,
        out_type=jax.ShapeDtypeStruct((NW, 128), jnp.float32),
        mesh=mesh,
        compiler_params=pltpu.CompilerParams(needs_layout_passes=False),
        scratch_types=[
            pltpu.VMEM((n_vars,), jnp.float32),
            pltpu.VMEM((L * rs,), jnp.float32),
            pltpu.VMEM((_B,), jnp.int32),
            pltpu.VMEM((_B,), jnp.int32),
            pltpu.VMEM((_B,), jnp.int32),
            pltpu.VMEM((_B,), jnp.int32),
            pltpu.VMEM((_B,), jnp.float32),
            pltpu.VMEM((_B,), jnp.float32),
            pltpu.VMEM((tpc,), jnp.float32),
            pltpu.VMEM((tpc,), jnp.int32),
            pltpu.VMEM((128,), jnp.int32),
            pltpu.VMEM((128,), jnp.float32),
            pltpu.SemaphoreType.DMA,
        ],
    )
    def k(pred_h, cidx_h, vidx_h, coeff_h, rhs_h, sense_h, bounds_h, out_h,
          probs_v, acc_v, cidx_b0, cidx_b1, vidx_b0, vidx_b1, coeff_b0,
          coeff_b1, rhs_b, sense_b, bounds_v, psum_b, sem):
        cidx_b = (cidx_b0, cidx_b1)
        vidx_b = (vidx_b0, vidx_b1)
        coeff_b = (coeff_b0, coeff_b1)
        cid = lax.axis_index("c")
        sid = lax.axis_index("s")
        wid = sid * NC + cid
        lane = lax.iota(jnp.int32, L)

        pltpu.sync_copy(pred_h, probs_v)
        pltpu.sync_copy(bounds_h, bounds_v)

        def sig_body(i, _):
            for u in range(_UNROLL):
                o = (i * _UNROLL + u) * L
                x = probs_v[pl.ds(o, L)]
                probs_v[pl.ds(o, L)] = 1.0 / (1.0 + jnp.exp(-x))
            return _

        lax.fori_loop(0, n_vars // (L * _UNROLL), sig_body, 0)

        start = bounds_v[pl.ds(wid, L)][0]
        end = bounds_v[pl.ds(wid + 1, L)][0]
        base_c = pl.multiple_of(wid * tpc, 16)

        zv = jnp.zeros((L,), jnp.float32)

        def z_body(i, _):
            acc_v[pl.ds(i * L, L)] = zv
            return _

        lax.fori_loop(0, (L * rs) // L, z_body, 0)

        a0 = jnp.bitwise_and(start, jnp.int32(-16))
        nblocks = (end - a0 + (_B - 1)) // _B
        lane_row = lane * rs

        def clamp_off(b):
            return pl.multiple_of(
                jnp.minimum(a0 + b * _B, jnp.int32(nnz - _B)), 16)

        def fetch(b, slot):
            off = clamp_off(b)
            pltpu.async_copy(cidx_h.at[pl.ds(off, _B)], cidx_b[slot], sem)
            pltpu.async_copy(vidx_h.at[pl.ds(off, _B)], vidx_b[slot], sem)
            pltpu.async_copy(coeff_h.at[pl.ds(off, _B)], coeff_b[slot], sem)

        def drain(slot):
            pltpu.make_async_copy(cidx_h.at[pl.ds(0, _B)], cidx_b[slot],
                                  sem).wait()
            pltpu.make_async_copy(vidx_h.at[pl.ds(0, _B)], vidx_b[slot],
                                  sem).wait()
            pltpu.make_async_copy(coeff_h.at[pl.ds(0, _B)], coeff_b[slot],
                                  sem).wait()

        def compute(b, slot):
            offc = clamp_off(b)
            offl = a0 + b * _B
            lo = jnp.maximum(start, offl)
            hi = jnp.minimum(end, offl + _B)
            interior = jnp.logical_and(lo == offc, hi == offc + _B)

            @pl.when(interior)
            def _():
                def step(s2, _2):
                    for u in range(_UNROLL):
                        o = (s2 * _UNROLL + u) * L
                        c = cidx_b[slot][pl.ds(o, L)]
                        v = vidx_b[slot][pl.ds(o, L)]
                        w = coeff_b[slot][pl.ds(o, L)]
                        p = plsc.load_gather(probs_v, [v])
                        slot_idx = lane_row + (c - base_c)
                        plsc.addupdate_scatter(acc_v, [slot_idx], w * p)
                    return _2

                lax.fori_loop(0, _STEPS // _UNROLL, step, 0)

            @pl.when(jnp.logical_not(interior))
            def _():
                def step(s2, _2):
                    c = cidx_b[slot][pl.ds(s2 * L, L)]
                    v = vidx_b[slot][pl.ds(s2 * L, L)]
                    w = coeff_b[slot][pl.ds(s2 * L, L)]
                    pos = offc + s2 * L + lane
                    m = (pos >= lo) & (pos < hi)
                    p = plsc.load_gather(probs_v, [v], mask=m)
                    slot_idx = jnp.where(m, lane_row + (c - base_c), 0)
                    plsc.addupdate_scatter(acc_v, [slot_idx], w * p, mask=m)
                    return _2

                lax.fori_loop(0, _STEPS, step, 0)

        @pl.when(nblocks > 0)
        def _():
            fetch(0, 0)

        def pair_body(m, carry):
            b0 = 2 * m
            b1 = b0 + 1

            drain(0)

            @pl.when(b1 < nblocks)
            def _():
                fetch(b1, 1)

            compute(b0, 0)

            @pl.when(b1 < nblocks)
            def _():
                drain(1)

                @pl.when(b1 + 1 < nblocks)
                def _():
                    fetch(b1 + 1, 0)

                compute(b1, 1)

            return carry

        lax.fori_loop(0, (nblocks + 1) // 2, pair_body, 0)

        pltpu.sync_copy(rhs_h.at[pl.ds(base_c, tpc)], rhs_b)
        pltpu.sync_copy(sense_h.at[pl.ds(base_c, tpc)], sense_b)

        def fin(q, ps):
            ax = acc_v[pl.ds(q * L, L)]
            for r in range(1, L):
                ax = ax + acc_v[pl.ds(r * rs + q * L, L)]
            d = ax - rhs_b[pl.ds(q * L, L)]
            ss = sense_b[pl.ds(q * L, L)]
            viol = jnp.where(
                ss == 1, jnp.maximum(d, 0.0),
                jnp.where(ss == 2, jnp.maximum(-d, 0.0),
                          jnp.where(ss == 3, jnp.abs(d),
                                    jnp.zeros((L,), jnp.float32))))
            return ps + viol

        psum = lax.fori_loop(0, tpc // L, fin, jnp.zeros((L,), jnp.float32))
        for q in range(128 // L):
            psum_b[pl.ds(q * L, L)] = psum if q == 0 else jnp.zeros(
                (L,), jnp.float32)
        pltpu.sync_copy(psum_b, out_h.at[wid])

    return k


def kernel(pred, constr_idx, var_idx, coeff, constr_rhs, constr_sense,
           n_vars, n_constrs):
    nv = pred.shape[0]
    ncs = constr_rhs.shape[0]
    nnz = constr_idx.shape[0]
    cidx = constr_idx.astype(jnp.int32)
    vidx = var_idx.astype(jnp.int32)
    sense = constr_sense.astype(jnp.int32)
    tpc = ncs // NW
    edges = jnp.arange(NW + 1, dtype=jnp.int32) * tpc
    bounds = jnp.searchsorted(cidx, edges, side="left",
                              method="compare_all").astype(jnp.int32)
    bounds128 = jnp.zeros((128,), jnp.int32).at[:NW + 1].set(bounds)
    partials = _build(nv, ncs, nnz)(
        pred.astype(jnp.float32), cidx, vidx, coeff.astype(jnp.float32),
        constr_rhs.astype(jnp.float32), sense, bounds128)
    return jnp.sum(partials) / ncs


# speedup vs baseline: 1.6330x; 1.6330x over previous
import functools

import jax
import jax.numpy as jnp
from jax import lax
from jax.experimental import pallas as pl
from jax.experimental.pallas import tpu as pltpu
from jax.experimental.pallas import tpu_sc as plsc

NC = 2
NS = 16
NW = NC * NS
L = 16

_B = 2048
_STEPS = _B // L
_UNROLL = 8


@functools.cache
def _build(n_vars, n_constrs, nnz):
    tpc = n_constrs // NW
    rs = tpc + 1
    mesh = plsc.VectorSubcoreMesh(core_axis_name="c", subcore_axis_name="s")

    @functools.partial(
        pl.kernel,
        out_type=jax.ShapeDtypeStruct((NW, 128), jnp.float32),
        mesh=mesh,
        compiler_params=pltpu.CompilerParams(needs_layout_passes=False),
        scratch_types=[
            pltpu.VMEM((n_vars,), jnp.float32),
            pltpu.VMEM((L * rs,), jnp.float32),
            pltpu.VMEM((_B,), jnp.int32),
            pltpu.VMEM((_B,), jnp.int32),
            pltpu.VMEM((_B,), jnp.int32),
            pltpu.VMEM((_B,), jnp.int32),
            pltpu.VMEM((_B,), jnp.float32),
            pltpu.VMEM((_B,), jnp.float32),
            pltpu.VMEM((tpc,), jnp.float32),
            pltpu.VMEM((tpc,), jnp.int32),
            pltpu.VMEM((128,), jnp.int32),
            pltpu.VMEM((128,), jnp.float32),
            pltpu.SemaphoreType.DMA,
        ],
    )
    def k(pred_h, cidx_h, vidx_h, coeff_h, rhs_h, sense_h, bounds_h, out_h,
          probs_v, acc_v, cidx_b0, cidx_b1, vidx_b0, vidx_b1, coeff_b0,
          coeff_b1, rhs_b, sense_b, bounds_v, psum_b, sem):
        cidx_b = (cidx_b0, cidx_b1)
        vidx_b = (vidx_b0, vidx_b1)
        coeff_b = (coeff_b0, coeff_b1)
        cid = lax.axis_index("c")
        sid = lax.axis_index("s")
        wid = sid * NC + cid
        lane = lax.iota(jnp.int32, L)

        pltpu.sync_copy(pred_h, probs_v)
        pltpu.sync_copy(bounds_h, bounds_v)

        def sig_body(i, _):
            for u in range(_UNROLL):
                o = (i * _UNROLL + u) * L
                x = probs_v[pl.ds(o, L)]
                probs_v[pl.ds(o, L)] = 1.0 / (1.0 + jnp.exp(-x))
            return _

        lax.fori_loop(0, n_vars // (L * _UNROLL), sig_body, 0)

        start = bounds_v[pl.ds(wid, L)][0]
        end = bounds_v[pl.ds(wid + 1, L)][0]
        base_c = pl.multiple_of(wid * tpc, 16)

        zv = jnp.zeros((L,), jnp.float32)

        def z_body(i, _):
            acc_v[pl.ds(i * L, L)] = zv
            return _

        lax.fori_loop(0, (L * rs) // L, z_body, 0)

        a0 = jnp.bitwise_and(start, jnp.int32(-16))
        nblocks = (end - a0 + (_B - 1)) // _B
        lane_row = lane * rs

        def clamp_off(b):
            return pl.multiple_of(
                jnp.minimum(a0 + b * _B, jnp.int32(nnz - _B)), 16)

        def fetch(b, slot):
            off = clamp_off(b)
            pltpu.async_copy(cidx_h.at[pl.ds(off, _B)], cidx_b[slot], sem)
            pltpu.async_copy(vidx_h.at[pl.ds(off, _B)], vidx_b[slot], sem)
            pltpu.async_copy(coeff_h.at[pl.ds(off, _B)], coeff_b[slot], sem)

        def drain(slot):
            pltpu.make_async_copy(cidx_h.at[pl.ds(0, _B)], cidx_b[slot],
                                  sem).wait()
            pltpu.make_async_copy(vidx_h.at[pl.ds(0, _B)], vidx_b[slot],
                                  sem).wait()
            pltpu.make_async_copy(coeff_h.at[pl.ds(0, _B)], coeff_b[slot],
                                  sem).wait()

        def compute(b, slot):
            offc = clamp_off(b)
            offl = a0 + b * _B
            lo = jnp.maximum(start, offl)
            hi = jnp.minimum(end, offl + _B)
            interior = jnp.logical_and(lo == offc, hi == offc + _B)

            @pl.when(interior)
            def _():
                def step(s2, _2):
                    for u in range(_UNROLL):
                        o = (s2 * _UNROLL + u) * L
                        c = cidx_b[slot][pl.ds(o, L)]
                        v = vidx_b[slot][pl.ds(o, L)]
                        w = coeff_b[slot][pl.ds(o, L)]
                        p = plsc.load_gather(probs_v, [v])
                        slot_idx = lane_row + (c - base_c)
                        plsc.addupdate_scatter(acc_v, [slot_idx], w * p)
                    return _2

                lax.fori_loop(0, _STEPS // _UNROLL, step, 0)

            @pl.when(jnp.logical_not(interior))
            def _():
                def step(s2, _2):
                    c = cidx_b[slot][pl.ds(s2 * L, L)]
                    v = vidx_b[slot][pl.ds(s2 * L, L)]
                    w = coeff_b[slot][pl.ds(s2 * L, L)]
                    pos = offc + s2 * L + lane
                    m = (pos >= lo) & (pos < hi)
                    p = plsc.load_gather(probs_v, [v], mask=m)
                    slot_idx = jnp.where(m, lane_row + (c - base_c), 0)
                    plsc.addupdate_scatter(acc_v, [slot_idx], w * p, mask=m)
                    return _2

                lax.fori_loop(0, _STEPS, step, 0)

        @pl.when(nblocks > 0)
        def _():
            fetch(0, 0)

        def pair_body(m, carry):
            b0 = 2 * m
            b1 = b0 + 1

            drain(0)

            @pl.when(b1 < nblocks)
            def _():
                fetch(b1, 1)

            compute(b0, 0)

            @pl.when(b1 < nblocks)
            def _():
                drain(1)

                @pl.when(b1 + 1 < nblocks)
                def _():
                    fetch(b1 + 1, 0)

                compute(b1, 1)

            return carry

        lax.fori_loop(0, (nblocks + 1) // 2, pair_body, 0)

        pltpu.sync_copy(rhs_h.at[pl.ds(base_c, tpc)], rhs_b)
        pltpu.sync_copy(sense_h.at[pl.ds(base_c, tpc)], sense_b)

        def fin(q, ps):
            ax = acc_v[pl.ds(q * L, L)]
            for r in range(1, L):
                ax = ax + acc_v[pl.ds(r * rs + q * L, L)]
            d = ax - rhs_b[pl.ds(q * L, L)]
            ss = sense_b[pl.ds(q * L, L)]
            viol = jnp.where(
                ss == 1, jnp.maximum(d, 0.0),
                jnp.where(ss == 2, jnp.maximum(-d, 0.0),
                          jnp.where(ss == 3, jnp.abs(d),
                                    jnp.zeros((L,), jnp.float32))))
            return ps + viol

        psum = lax.fori_loop(0, tpc // L, fin, jnp.zeros((L,), jnp.float32))
        for q in range(128 // L):
            psum_b[pl.ds(q * L, L)] = psum if q == 0 else jnp.zeros(
                (L,), jnp.float32)
        pltpu.sync_copy(psum_b, out_h.at[wid])

    return k


def kernel(pred, constr_idx, var_idx, coeff, constr_rhs, constr_sense,
           n_vars, n_constrs):
    nv = pred.shape[0]
    ncs = constr_rhs.shape[0]
    nnz = constr_idx.shape[0]
    cidx = constr_idx.astype(jnp.int32)
    vidx = var_idx.astype(jnp.int32)
    sense = constr_sense.astype(jnp.int32)
    tpc = ncs // NW
    edges = jnp.arange(NW + 1, dtype=jnp.int32) * tpc
    bounds = jnp.searchsorted(cidx, edges, side="left").astype(jnp.int32)
    bounds128 = jnp.zeros((128,), jnp.int32).at[:NW + 1].set(bounds)
    partials = _build(nv, ncs, nnz)(
        pred.astype(jnp.float32), cidx, vidx, coeff.astype(jnp.float32),
        constr_rhs.astype(jnp.float32), sense, bounds128)
    return jnp.sum(partials) / ncs
